# Initial kernel scaffold; baseline (speedup 1.0000x reference)
import jax
import jax.numpy as jnp
from jax.experimental import pallas as pl


def kernel(inputs, table):
    raise NotImplementedError("write your pallas kernel here")


# speedup vs baseline: 135.7359x; 135.7359x over previous
import functools

import jax
import jax.numpy as jnp
from jax import lax
from jax.experimental import pallas as pl
from jax.experimental.pallas import tpu as pltpu
from jax.experimental.pallas import tpu_sc as plsc

_NC = 2
_NS = 16
_NW = _NC * _NS

_N = 16384 * 200
_PER_W = _N // _NW
_CHUNK = 12800
_NCHUNK = _PER_W // _CHUNK


def _sc_gather(table, idx_flat):
    mesh = plsc.VectorSubcoreMesh(core_axis_name="c", subcore_axis_name="s")

    @functools.partial(
        pl.kernel,
        mesh=mesh,
        out_type=jax.ShapeDtypeStruct((_N,), jnp.int32),
        scratch_types=[
            pltpu.VMEM((_CHUNK,), jnp.int32),
            pltpu.VMEM((_CHUNK,), jnp.int32),
            pltpu.SemaphoreType.DMA,
        ],
    )
    def k(table_hbm, idx_hbm, out_hbm, idx_v, val_v, sem):
        wid = lax.axis_index("s") * _NC + lax.axis_index("c")
        base = wid * _PER_W

        def body(i, carry):
            off = base + i * _CHUNK
            pltpu.sync_copy(idx_hbm.at[pl.ds(off, _CHUNK)], idx_v)
            pltpu.async_copy(table_hbm.at[idx_v], val_v, sem).wait()
            pltpu.sync_copy(val_v, out_hbm.at[pl.ds(off, _CHUNK)])
            return carry

        lax.fori_loop(0, _NCHUNK, body, 0)

    return k(table, idx_flat)


def kernel(inputs, table):
    out = _sc_gather(table, inputs.reshape(-1))
    return out.reshape(inputs.shape)


# speedup vs baseline: 138.5235x; 1.0205x over previous
import functools

import jax
import jax.numpy as jnp
from jax import lax
from jax.experimental import pallas as pl
from jax.experimental.pallas import tpu as pltpu
from jax.experimental.pallas import tpu_sc as plsc

_NC = 2
_NS = 16
_NW = _NC * _NS

_N = 16384 * 200
_PER_W = _N // _NW
_CHUNK = 12800
_NCHUNK = _PER_W // _CHUNK
_B = 3


def _sc_gather(table, idx_flat):
    mesh = plsc.VectorSubcoreMesh(core_axis_name="c", subcore_axis_name="s")

    scratch = (
        [pltpu.VMEM((_CHUNK,), jnp.int32) for _ in range(2 * _B)]
        + [pltpu.SemaphoreType.DMA for _ in range(3 * _B)]
    )

    @functools.partial(
        pl.kernel,
        mesh=mesh,
        out_type=jax.ShapeDtypeStruct((_N,), jnp.int32),
        scratch_types=scratch,
    )
    def k(table_hbm, idx_hbm, out_hbm, *refs):
        idx_v = refs[0:_B]
        val_v = refs[_B:2 * _B]
        s_in = refs[2 * _B:2 * _B + _B]
        s_g = refs[2 * _B + _B:2 * _B + 2 * _B]
        s_o = refs[2 * _B + 2 * _B:2 * _B + 3 * _B]

        wid = lax.axis_index("s") * _NC + lax.axis_index("c")
        base = wid * _PER_W

        def off(i):
            return base + i * _CHUNK

        in_d = [None] * _NCHUNK
        g_d = [None] * _NCHUNK
        o_d = [None] * _NCHUNK

        for i in range(min(_B, _NCHUNK)):
            in_d[i] = pltpu.async_copy(
                idx_hbm.at[pl.ds(off(i), _CHUNK)], idx_v[i % _B], s_in[i % _B])

        for i in range(_NCHUNK):
            b = i % _B
            in_d[i].wait()
            if i >= _B:
                o_d[i - _B].wait()
            g_d[i] = pltpu.async_copy(table_hbm.at[idx_v[b]], val_v[b], s_g[b])
            g_d[i].wait()
            o_d[i] = pltpu.async_copy(
                val_v[b], out_hbm.at[pl.ds(off(i), _CHUNK)], s_o[b])
            if i + _B < _NCHUNK:
                in_d[i + _B] = pltpu.async_copy(
                    idx_hbm.at[pl.ds(off(i + _B), _CHUNK)], idx_v[b], s_in[b])

        for i in range(max(0, _NCHUNK - _B), _NCHUNK):
            o_d[i].wait()

    return k(table, idx_flat)


def kernel(inputs, table):
    out = _sc_gather(table, inputs.reshape(-1))
    return out.reshape(inputs.shape)


# speedup vs baseline: 223.9734x; 1.6169x over previous
import functools

import jax
import jax.numpy as jnp
from jax import lax
from jax.experimental import pallas as pl
from jax.experimental.pallas import tpu as pltpu
from jax.experimental.pallas import tpu_sc as plsc

_NC = 2
_NS = 16
_NW = _NC * _NS

_N = 16384 * 200
_PER_W = _N // _NW
_CHUNK = 10240
_NCHUNK = _PER_W // _CHUNK
_B = 3

_V = 1000000
_NVSUB = _V // _CHUNK
_VTAIL = _V - _NVSUB * _CHUNK


def _sc_gather(table, idx_flat):
    mesh = plsc.VectorSubcoreMesh(core_axis_name="c", subcore_axis_name="s")

    scratch = (
        [pltpu.VMEM_SHARED((_V,), jnp.int32)]
        + [pltpu.VMEM((_CHUNK,), jnp.int32) for _ in range(2 * _B)]
        + [pltpu.SemaphoreType.DMA for _ in range(3 * _B + 1)]
    )

    @functools.partial(
        pl.kernel,
        mesh=mesh,
        out_type=jax.ShapeDtypeStruct((_N,), jnp.int32),
        scratch_types=scratch,
    )
    def k(table_hbm, idx_hbm, out_hbm, tab_s, *refs):
        idx_v = refs[0:_B]
        val_v = refs[_B:2 * _B]
        s_in = refs[2 * _B:3 * _B]
        s_g = refs[3 * _B:4 * _B]
        s_o = refs[4 * _B:5 * _B]
        s_stage = refs[5 * _B]

        cid = lax.axis_index("c")
        sid = lax.axis_index("s")
        wid = sid * _NC + cid
        base = wid * _PER_W

        def _stage_one(off_w, size):
            pltpu.async_copy(
                table_hbm.at[pl.ds(off_w, size)],
                val_v[0].at[pl.ds(0, size)], s_stage).wait()
            pltpu.async_copy(
                val_v[0].at[pl.ds(0, size)],
                tab_s.at[pl.ds(off_w, size)], s_stage).wait()

        for t in range((_NVSUB + 1 + _NS - 1) // _NS):
            j = t * _NS + sid

            @pl.when(j < _NVSUB)
            def _full(j=j):
                _stage_one(j * _CHUNK, _CHUNK)

            @pl.when(j == _NVSUB)
            def _tail(j=j):
                _stage_one(_NVSUB * _CHUNK, _VTAIL)

        plsc.subcore_barrier()

        def off(i):
            return base + i * _CHUNK

        in_d = [None] * _NCHUNK
        g_d = [None] * _NCHUNK
        o_d = [None] * _NCHUNK

        for i in range(min(_B, _NCHUNK)):
            in_d[i] = pltpu.async_copy(
                idx_hbm.at[pl.ds(off(i), _CHUNK)], idx_v[i % _B], s_in[i % _B])

        for i in range(_NCHUNK):
            b = i % _B
            in_d[i].wait()
            if i >= _B:
                o_d[i - _B].wait()
            g_d[i] = pltpu.async_copy(tab_s.at[idx_v[b]], val_v[b], s_g[b])
            g_d[i].wait()
            o_d[i] = pltpu.async_copy(
                val_v[b], out_hbm.at[pl.ds(off(i), _CHUNK)], s_o[b])
            if i + _B < _NCHUNK:
                in_d[i + _B] = pltpu.async_copy(
                    idx_hbm.at[pl.ds(off(i + _B), _CHUNK)], idx_v[b], s_in[b])

        for i in range(max(0, _NCHUNK - _B), _NCHUNK):
            o_d[i].wait()

    return k(table, idx_flat)


def kernel(inputs, table):
    out = _sc_gather(table, inputs.reshape(-1))
    return out.reshape(inputs.shape)


# speedup vs baseline: 543.6646x; 2.4274x over previous
import functools

import jax
import jax.numpy as jnp
from jax import lax
from jax.experimental import pallas as pl
from jax.experimental.pallas import tpu as pltpu
from jax.experimental.pallas import tpu_sc as plsc

_NC = 2
_NS = 16
_NW = _NC * _NS

_N = 16384 * 200
_PER_W = _N // _NW
_CHUNK = 10240
_NCHUNK = _PER_W // _CHUNK
_B = 3

_V = 1000000
_NVSUB = _V // _CHUNK
_VTAIL = _V - _NVSUB * _CHUNK
_NST = 6


def _sc_gather(table, idx_flat):
    mesh = plsc.VectorSubcoreMesh(core_axis_name="c", subcore_axis_name="s")

    scratch = (
        [pltpu.VMEM_SHARED((_V,), jnp.int32)]
        + [pltpu.VMEM((_CHUNK,), jnp.int32) for _ in range(2 * _B)]
        + [pltpu.SemaphoreType.DMA for _ in range(3 * _B + 2)]
    )

    @functools.partial(
        pl.kernel,
        mesh=mesh,
        out_type=jax.ShapeDtypeStruct((_N,), jnp.int32),
        scratch_types=scratch,
    )
    def k(table_hbm, idx_hbm, out_hbm, tab_s, *refs):
        idx_v = refs[0:_B]
        val_v = refs[_B:2 * _B]
        s_in = refs[2 * _B:3 * _B]
        s_g = refs[3 * _B:4 * _B]
        s_o = refs[4 * _B:5 * _B]
        s_sta = refs[5 * _B]
        s_stb = refs[5 * _B + 1]

        cid = lax.axis_index("c")
        sid = lax.axis_index("s")
        wid = sid * _NC + cid
        base = wid * _PER_W

        def off(i):
            return base + i * _CHUNK

        in_d = [None] * _NCHUNK
        g_d = [None] * _NCHUNK
        o_d = [None] * _NCHUNK

        for i in range(min(_B, _NCHUNK)):
            in_d[i] = pltpu.async_copy(
                idx_hbm.at[pl.ds(off(i), _CHUNK)], idx_v[i % _B], s_in[i % _B])

        bounce = (val_v[0], val_v[1])
        s_bin = (s_g[0], s_g[1])
        s_bout = (s_sta, s_stb)

        def st_off(t):
            return (t * _NS + sid) * _CHUNK

        st_in = [None] * _NST
        st_out = [None] * _NST
        for t in range(2):
            st_in[t] = pltpu.async_copy(
                table_hbm.at[pl.ds(st_off(t), _CHUNK)], bounce[t], s_bin[t])
        for t in range(_NST):
            b = t % 2
            st_in[t].wait()
            st_out[t] = pltpu.async_copy(
                bounce[b], tab_s.at[pl.ds(st_off(t), _CHUNK)], s_bout[b])
            if t + 2 < _NST:
                st_out[t].wait()
                st_in[t + 2] = pltpu.async_copy(
                    table_hbm.at[pl.ds(st_off(t + 2), _CHUNK)], bounce[b],
                    s_bin[b])
        st_out[_NST - 2].wait()
        st_out[_NST - 1].wait()

        @pl.when(sid == 0)
        def _last_full():
            pltpu.async_copy(
                table_hbm.at[pl.ds(96 * _CHUNK, _CHUNK)], bounce[0],
                s_bin[0]).wait()
            pltpu.async_copy(
                bounce[0], tab_s.at[pl.ds(96 * _CHUNK, _CHUNK)],
                s_bout[0]).wait()

        @pl.when(sid == 1)
        def _tail():
            pltpu.async_copy(
                table_hbm.at[pl.ds(_NVSUB * _CHUNK, _VTAIL)],
                bounce[1].at[pl.ds(0, _VTAIL)], s_bin[1]).wait()
            pltpu.async_copy(
                bounce[1].at[pl.ds(0, _VTAIL)],
                tab_s.at[pl.ds(_NVSUB * _CHUNK, _VTAIL)], s_bout[1]).wait()

        plsc.subcore_barrier()

        in_d[0].wait()
        g_d[0] = pltpu.async_copy(tab_s.at[idx_v[0]], val_v[0], s_g[0])
        for i in range(_NCHUNK):
            b = i % _B
            nb = (i + 1) % _B
            if i + 1 < _NCHUNK:
                in_d[i + 1].wait()
                if i + 1 >= _B:
                    o_d[i + 1 - _B].wait()
                g_d[i + 1] = pltpu.async_copy(
                    tab_s.at[idx_v[nb]], val_v[nb], s_g[nb])
            g_d[i].wait()
            o_d[i] = pltpu.async_copy(
                val_v[b], out_hbm.at[pl.ds(off(i), _CHUNK)], s_o[b])
            if i + _B < _NCHUNK:
                in_d[i + _B] = pltpu.async_copy(
                    idx_hbm.at[pl.ds(off(i + _B), _CHUNK)], idx_v[b], s_in[b])

        for i in range(max(0, _NCHUNK - _B), _NCHUNK):
            o_d[i].wait()

    return k(table, idx_flat)


def kernel(inputs, table):
    raw = (inputs.T.reshape(25, 8, 128, 128)
           .transpose(0, 2, 1, 3).reshape(-1))
    out_raw = _sc_gather(table, raw)
    out_t = (out_raw.reshape(25, 128, 8, 128)
             .transpose(0, 2, 1, 3).reshape(200, 16384))
    return out_t.T


# speedup vs baseline: 544.8016x; 1.0021x over previous
import functools

import jax
import jax.numpy as jnp
from jax import lax
from jax.experimental import pallas as pl
from jax.experimental.pallas import tpu as pltpu
from jax.experimental.pallas import tpu_sc as plsc

_NC = 2
_NS = 16
_NW = _NC * _NS

_N = 16384 * 200
_PER_W = _N // _NW
_CHUNK = 10240
_NCHUNK = _PER_W // _CHUNK
_B = 3

_V = 1000000
_NVSUB = _V // _CHUNK
_VTAIL = _V - _NVSUB * _CHUNK
_NST = 6


def _sc_gather(table, idx_flat):
    mesh = plsc.VectorSubcoreMesh(core_axis_name="c", subcore_axis_name="s")

    scratch = (
        [pltpu.VMEM_SHARED((_V,), jnp.int32)]
        + [pltpu.VMEM((_CHUNK,), jnp.int32) for _ in range(2 * _B)]
        + [pltpu.SemaphoreType.DMA for _ in range(3 * _B + 2)]
    )

    @functools.partial(
        pl.kernel,
        mesh=mesh,
        out_type=jax.ShapeDtypeStruct((_N,), jnp.int32),
        scratch_types=scratch,
    )
    def k(table_hbm, idx_hbm, out_hbm, tab_s, *refs):
        idx_v = refs[0:_B]
        val_v = refs[_B:2 * _B]
        s_in = refs[2 * _B:3 * _B]
        s_g = refs[3 * _B:4 * _B]
        s_o = refs[4 * _B:5 * _B]
        s_sta = refs[5 * _B]
        s_stb = refs[5 * _B + 1]

        cid = lax.axis_index("c")
        sid = lax.axis_index("s")
        wid = sid * _NC + cid
        base = wid * _PER_W

        def off(i):
            return base + i * _CHUNK

        in_d = [None] * _NCHUNK
        g_d = [None] * _NCHUNK
        o_d = [None] * _NCHUNK

        for i in range(min(_B, _NCHUNK)):
            in_d[i] = pltpu.async_copy(
                idx_hbm.at[pl.ds(off(i), _CHUNK)], idx_v[i % _B], s_in[i % _B])

        bounce = (val_v[0], val_v[1])
        s_bin = (s_g[0], s_g[1])
        s_bout = (s_sta, s_stb)

        def st_off(t):
            return (t * _NS + sid) * _CHUNK

        st_in = [None] * _NST
        st_out = [None] * _NST
        for t in range(2):
            st_in[t] = pltpu.async_copy(
                table_hbm.at[pl.ds(st_off(t), _CHUNK)], bounce[t], s_bin[t])
        for t in range(_NST):
            b = t % 2
            st_in[t].wait()
            st_out[t] = pltpu.async_copy(
                bounce[b], tab_s.at[pl.ds(st_off(t), _CHUNK)], s_bout[b])
            if t + 2 < _NST:
                st_out[t].wait()
                st_in[t + 2] = pltpu.async_copy(
                    table_hbm.at[pl.ds(st_off(t + 2), _CHUNK)], bounce[b],
                    s_bin[b])
        st_out[_NST - 2].wait()
        st_out[_NST - 1].wait()

        @pl.when(sid == 0)
        def _last_full():
            pltpu.async_copy(
                table_hbm.at[pl.ds(96 * _CHUNK, _CHUNK)], bounce[0],
                s_bin[0]).wait()
            pltpu.async_copy(
                bounce[0], tab_s.at[pl.ds(96 * _CHUNK, _CHUNK)],
                s_bout[0]).wait()

        @pl.when(sid == 1)
        def _tail():
            pltpu.async_copy(
                table_hbm.at[pl.ds(_NVSUB * _CHUNK, _VTAIL)],
                bounce[1].at[pl.ds(0, _VTAIL)], s_bin[1]).wait()
            pltpu.async_copy(
                bounce[1].at[pl.ds(0, _VTAIL)],
                tab_s.at[pl.ds(_NVSUB * _CHUNK, _VTAIL)], s_bout[1]).wait()

        plsc.subcore_barrier()

        for j in range(2):
            in_d[j].wait()
            g_d[j] = pltpu.async_copy(tab_s.at[idx_v[j]], val_v[j], s_g[j])
        for i in range(_NCHUNK):
            b = i % _B
            if i + 2 < _NCHUNK:
                nb = (i + 2) % _B
                in_d[i + 2].wait()
                if i + 2 >= _B:
                    o_d[i + 2 - _B].wait()
                g_d[i + 2] = pltpu.async_copy(
                    tab_s.at[idx_v[nb]], val_v[nb], s_g[nb])
            g_d[i].wait()
            o_d[i] = pltpu.async_copy(
                val_v[b], out_hbm.at[pl.ds(off(i), _CHUNK)], s_o[b])
            if i + _B < _NCHUNK:
                in_d[i + _B] = pltpu.async_copy(
                    idx_hbm.at[pl.ds(off(i + _B), _CHUNK)], idx_v[b], s_in[b])

        for i in range(max(0, _NCHUNK - _B), _NCHUNK):
            o_d[i].wait()

    return k(table, idx_flat)


def kernel(inputs, table):
    raw = (inputs.T.reshape(25, 8, 128, 128)
           .transpose(0, 2, 1, 3).reshape(-1))
    out_raw = _sc_gather(table, raw)
    out_t = (out_raw.reshape(25, 128, 8, 128)
             .transpose(0, 2, 1, 3).reshape(200, 16384))
    return out_t.T


# speedup vs baseline: 555.8728x; 1.0203x over previous
import functools

import jax
import jax.numpy as jnp
from jax import lax
from jax.experimental import pallas as pl
from jax.experimental.pallas import tpu as pltpu
from jax.experimental.pallas import tpu_sc as plsc

_NC = 2
_NS = 16
_NW = _NC * _NS

_N = 16384 * 200
_PER_W = _N // _NW
_CHUNK = 10240
_NCHUNK = _PER_W // _CHUNK
_B = 3

_V = 1000000
_NST = 6
_STMAIN = _NST * _NS * _CHUNK
_VTAIL = (_V - _STMAIN) // 8


def _sc_gather(table, idx_flat):
    mesh = plsc.VectorSubcoreMesh(core_axis_name="c", subcore_axis_name="s")

    scratch = (
        [pltpu.VMEM_SHARED((_V,), jnp.int32)]
        + [pltpu.VMEM((_CHUNK,), jnp.int32) for _ in range(2 * _B)]
        + [pltpu.SemaphoreType.DMA for _ in range(3 * _B + 3)]
    )

    @functools.partial(
        pl.kernel,
        mesh=mesh,
        out_type=jax.ShapeDtypeStruct((_N,), jnp.int32),
        scratch_types=scratch,
    )
    def k(table_hbm, idx_hbm, out_hbm, tab_s, *refs):
        idx_v = refs[0:_B]
        val_v = refs[_B:2 * _B]
        s_in = refs[2 * _B:3 * _B]
        s_g = refs[3 * _B:4 * _B]
        s_o = refs[4 * _B:5 * _B]
        s_sta = refs[5 * _B]
        s_stb = refs[5 * _B + 1]
        s_stc = refs[5 * _B + 2]

        cid = lax.axis_index("c")
        sid = lax.axis_index("s")
        wid = sid * _NC + cid
        base = wid * _PER_W

        def off(i):
            return base + i * _CHUNK

        in_d = [None] * _NCHUNK
        g_d = [None] * _NCHUNK
        o_d = [None] * _NCHUNK

        for i in range(min(_B, _NCHUNK)):
            in_d[i] = pltpu.async_copy(
                idx_hbm.at[pl.ds(off(i), _CHUNK)], idx_v[i % _B], s_in[i % _B])

        bounce = (val_v[0], val_v[1], val_v[2])
        s_bin = (s_g[0], s_g[1], s_g[2])
        s_bout = (s_sta, s_stb, s_stc)

        def st_off(t):
            return (t * _NS + sid) * _CHUNK

        st_in = [None] * _NST
        st_out = [None] * _NST
        for t in range(3):
            st_in[t] = pltpu.async_copy(
                table_hbm.at[pl.ds(st_off(t), _CHUNK)], bounce[t], s_bin[t])
        for t in range(_NST):
            b = t % 3
            st_in[t].wait()
            st_out[t] = pltpu.async_copy(
                bounce[b], tab_s.at[pl.ds(st_off(t), _CHUNK)], s_bout[b])
            if t + 3 < _NST:
                st_out[t].wait()
                st_in[t + 3] = pltpu.async_copy(
                    table_hbm.at[pl.ds(st_off(t + 3), _CHUNK)], bounce[b],
                    s_bin[b])
        for t in range(_NST - 3, _NST):
            st_out[t].wait()

        @pl.when(sid < 8)
        def _tail():
            toff = _STMAIN + sid * _VTAIL
            pltpu.async_copy(
                table_hbm.at[pl.ds(toff, _VTAIL)],
                bounce[0].at[pl.ds(0, _VTAIL)], s_bin[0]).wait()
            pltpu.async_copy(
                bounce[0].at[pl.ds(0, _VTAIL)],
                tab_s.at[pl.ds(toff, _VTAIL)], s_bout[0]).wait()

        plsc.subcore_barrier()

        for j in range(2):
            in_d[j].wait()
            g_d[j] = pltpu.async_copy(tab_s.at[idx_v[j]], val_v[j], s_g[j])
        for i in range(_NCHUNK):
            b = i % _B
            if i + 2 < _NCHUNK:
                nb = (i + 2) % _B
                in_d[i + 2].wait()
                if i + 2 >= _B:
                    o_d[i + 2 - _B].wait()
                g_d[i + 2] = pltpu.async_copy(
                    tab_s.at[idx_v[nb]], val_v[nb], s_g[nb])
            g_d[i].wait()
            o_d[i] = pltpu.async_copy(
                val_v[b], out_hbm.at[pl.ds(off(i), _CHUNK)], s_o[b])
            if i + _B < _NCHUNK:
                in_d[i + _B] = pltpu.async_copy(
                    idx_hbm.at[pl.ds(off(i + _B), _CHUNK)], idx_v[b], s_in[b])

        for i in range(max(0, _NCHUNK - _B), _NCHUNK):
            o_d[i].wait()

    return k(table, idx_flat)


def kernel(inputs, table):
    raw = (inputs.T.reshape(25, 8, 128, 128)
           .transpose(0, 2, 1, 3).reshape(-1))
    out_raw = _sc_gather(table, raw)
    out_t = (out_raw.reshape(25, 128, 8, 128)
             .transpose(0, 2, 1, 3).reshape(200, 16384))
    return out_t.T
